# baseline (speedup 1.0000x reference)
import jax
import jax.numpy as jnp
from jax.experimental import pallas as pl


def _mm_body(p_ref, x_ref, o_ref):
    o_ref[...] = jnp.dot(p_ref[...], x_ref[...],
                         preferred_element_type=jnp.float32)


def kernel(x_coarse, P):
    N, Nc = P.shape
    F = x_coarse.shape[1]
    BM = 2000
    return pl.pallas_call(
        _mm_body,
        grid=(N // BM,),
        in_specs=[
            pl.BlockSpec((BM, Nc), lambda i: (i, 0)),
            pl.BlockSpec((Nc, F), lambda i: (0, 0)),
        ],
        out_specs=pl.BlockSpec((BM, F), lambda i: (i, 0)),
        out_shape=jax.ShapeDtypeStruct((N, F), jnp.float32),
    )(P, x_coarse)
